# Initial kernel scaffold; baseline (speedup 1.0000x reference)
import jax
import jax.numpy as jnp
from jax.experimental import pallas as pl


def kernel(head, relation, tail, entity_emb, relation_emb):
    raise NotImplementedError("write your pallas kernel here")


# speedup vs baseline: 1.2900x; 1.2900x over previous
import functools

import jax
import jax.numpy as jnp
from jax import lax
from jax.experimental import pallas as pl
from jax.experimental.pallas import tpu as pltpu
from jax.experimental.pallas import tpu_sc as plsc

D = 128
L = 16
NW = 32
CH = 128


def _tsvec_sc(batch):
  b_per_w = batch // NW
  n_chunks = b_per_w // CH
  mesh = plsc.VectorSubcoreMesh(core_axis_name="c", subcore_axis_name="s")

  @functools.partial(
      pl.kernel,
      mesh=mesh,
      compiler_params=pltpu.CompilerParams(needs_layout_passes=False),
      out_type=jax.ShapeDtypeStruct((NW, b_per_w), jnp.float32),
      scratch_types=[
          pltpu.VMEM((n_chunks, CH), jnp.int32),
          pltpu.VMEM((n_chunks, CH), jnp.int32),
          pltpu.VMEM((n_chunks, CH), jnp.int32),
          pltpu.VMEM((2, CH, D), jnp.float32),
          pltpu.VMEM((2, CH, D), jnp.float32),
          pltpu.VMEM((2, CH, D), jnp.float32),
          pltpu.VMEM((b_per_w,), jnp.float32),
          pltpu.SemaphoreType.DMA((2,)),
      ],
  )
  def k(head_hbm, rel_hbm, tail_hbm, ent_hbm, relemb_hbm, out_hbm,
        hidx, ridx, tidx, hbuf, rbuf, tbuf, obuf, sems):
    wid = lax.axis_index("s") * 2 + lax.axis_index("c")

    pltpu.sync_copy(head_hbm.at[wid], hidx)
    pltpu.sync_copy(rel_hbm.at[wid], ridx)
    pltpu.sync_copy(tail_hbm.at[wid], tidx)

    def fire(ci, slot):
      pltpu.async_copy(ent_hbm.at[hidx.at[ci]], hbuf.at[slot], sems.at[slot])
      pltpu.async_copy(relemb_hbm.at[ridx.at[ci]], rbuf.at[slot], sems.at[slot])
      pltpu.async_copy(ent_hbm.at[tidx.at[ci]], tbuf.at[slot], sems.at[slot])

    def drain(ci, slot):
      pltpu.make_async_copy(ent_hbm.at[hidx.at[ci]], hbuf.at[slot],
                            sems.at[slot]).wait()
      pltpu.make_async_copy(relemb_hbm.at[ridx.at[ci]], rbuf.at[slot],
                            sems.at[slot]).wait()
      pltpu.make_async_copy(ent_hbm.at[tidx.at[ci]], tbuf.at[slot],
                            sems.at[slot]).wait()

    fire(0, 0)
    for ci in range(n_chunks):
      slot = ci % 2
      if ci + 1 < n_chunks:
        fire(ci + 1, 1 - slot)
      drain(ci, slot)

      lanes = lax.iota(jnp.int32, L)

      def blk_body(blk, _):
        base = blk * L
        outv = jnp.zeros((L,), jnp.float32)
        for j in range(L):
          acc = jnp.zeros((L,), jnp.float32)
          for c in range(D // L):
            hv = hbuf[slot, base + j, pl.ds(c * L, L)]
            rv = rbuf[slot, base + j, pl.ds(c * L, L)]
            tv = tbuf[slot, base + j, pl.ds(c * L, L)]
            d = (hv + rv) - tv
            acc = acc + d * d
          outv = jnp.where(lanes == j, jnp.sum(acc), outv)
        obuf[pl.ds(ci * CH + base, L)] = outv
        return 0

      lax.fori_loop(0, CH // L, blk_body, 0)

    for v in range(b_per_w // L):
      x = obuf[pl.ds(v * L, L)]
      i = plsc.bitcast(x, jnp.int32)
      i = jnp.int32(0x5F3759DF) - (i >> 1)
      y = plsc.bitcast(i, jnp.float32)
      half_x = 0.5 * x
      for _ in range(3):
        y = y * (1.5 - half_x * y * y)
      obuf[pl.ds(v * L, L)] = -(x * y)

    pltpu.sync_copy(obuf, out_hbm.at[wid])

  return k


def kernel(head, relation, tail, entity_emb, relation_emb):
  batch = head.shape[0]
  b_per_w = batch // NW
  n_chunks = b_per_w // CH
  head_r = head.reshape(NW, n_chunks, CH)
  rel_r = relation.reshape(NW, n_chunks, CH)
  tail_r = tail.reshape(NW, n_chunks, CH)
  out = _tsvec_sc(batch)(head_r, rel_r, tail_r, entity_emb, relation_emb)
  return out.reshape(batch)


# speedup vs baseline: 1.3698x; 1.0619x over previous
import functools

import jax
import jax.numpy as jnp
from jax import lax
from jax.experimental import pallas as pl
from jax.experimental.pallas import tpu as pltpu
from jax.experimental.pallas import tpu_sc as plsc

D = 128
L = 16
NW = 32
CH = 128
NSLOT = 3


def _tsvec_sc(batch):
  b_per_w = batch // NW
  n_chunks = b_per_w // CH
  mesh = plsc.VectorSubcoreMesh(core_axis_name="c", subcore_axis_name="s")

  @functools.partial(
      pl.kernel,
      mesh=mesh,
      compiler_params=pltpu.CompilerParams(needs_layout_passes=False),
      out_type=jax.ShapeDtypeStruct((NW, b_per_w), jnp.float32),
      scratch_types=[
          pltpu.VMEM((n_chunks, CH), jnp.int32),
          pltpu.VMEM((n_chunks, CH), jnp.int32),
          pltpu.VMEM((n_chunks, CH), jnp.int32),
          pltpu.VMEM((NSLOT, CH, D), jnp.float32),
          pltpu.VMEM((NSLOT, CH, D), jnp.float32),
          pltpu.VMEM((b_per_w,), jnp.float32),
          pltpu.SemaphoreType.DMA((NSLOT,)),
          pltpu.SemaphoreType.DMA((NSLOT,)),
          pltpu.SemaphoreType.DMA((NSLOT,)),
      ],
  )
  def k(head_hbm, rel_hbm, tail_hbm, ent_hbm, relemb_hbm, out_hbm,
        hidx, ridx, tidx, hbuf, tbuf, obuf, hsem, tsem, rsem):
    wid = lax.axis_index("s") * 2 + lax.axis_index("c")

    pltpu.sync_copy(head_hbm.at[wid], hidx)
    pltpu.sync_copy(rel_hbm.at[wid], ridx)
    pltpu.sync_copy(tail_hbm.at[wid], tidx)

    def fire_ht(ci):
      slot = ci % NSLOT
      pltpu.async_copy(ent_hbm.at[hidx.at[ci]], hbuf.at[slot], hsem.at[slot])
      pltpu.async_copy(ent_hbm.at[tidx.at[ci]], tbuf.at[slot], tsem.at[slot])

    def wait_h(ci):
      slot = ci % NSLOT
      pltpu.make_async_copy(ent_hbm.at[hidx.at[ci]], hbuf.at[slot],
                            hsem.at[slot]).wait()

    def fire_radd(ci):
      slot = ci % NSLOT
      pltpu.async_copy(relemb_hbm.at[ridx.at[ci]], hbuf.at[slot],
                       rsem.at[slot], add=True)

    def wait_tr(ci):
      slot = ci % NSLOT
      pltpu.make_async_copy(ent_hbm.at[tidx.at[ci]], tbuf.at[slot],
                            tsem.at[slot]).wait()
      pltpu.make_async_copy(relemb_hbm.at[ridx.at[ci]], hbuf.at[slot],
                            rsem.at[slot]).wait()

    fire_ht(0)
    if n_chunks > 1:
      fire_ht(1)
    wait_h(0)
    fire_radd(0)

    for ci in range(n_chunks):
      slot = ci % NSLOT
      if ci + 2 < n_chunks:
        fire_ht(ci + 2)
      if ci + 1 < n_chunks:
        wait_h(ci + 1)
        fire_radd(ci + 1)
      wait_tr(ci)

      lanes = lax.iota(jnp.int32, L)

      @plsc.parallel_loop(0, CH // L)
      def blk_body(blk):
        base = blk * L
        outv = jnp.zeros((L,), jnp.float32)
        for j in range(L):
          acc = jnp.zeros((L,), jnp.float32)
          for c in range(D // L):
            hv = hbuf[slot, base + j, pl.ds(c * L, L)]
            tv = tbuf[slot, base + j, pl.ds(c * L, L)]
            d = hv - tv
            acc = acc + d * d
          outv = jnp.where(lanes == j, jnp.sum(acc), outv)
        obuf[pl.ds(ci * CH + base, L)] = outv

    for v in range(b_per_w // L):
      x = obuf[pl.ds(v * L, L)]
      i = plsc.bitcast(x, jnp.int32)
      i = jnp.int32(0x5F3759DF) - (i >> 1)
      y = plsc.bitcast(i, jnp.float32)
      half_x = 0.5 * x
      for _ in range(3):
        y = y * (1.5 - half_x * y * y)
      obuf[pl.ds(v * L, L)] = -(x * y)

    pltpu.sync_copy(obuf, out_hbm.at[wid])

  return k


def kernel(head, relation, tail, entity_emb, relation_emb):
  batch = head.shape[0]
  b_per_w = batch // NW
  n_chunks = b_per_w // CH
  head_r = head.reshape(NW, n_chunks, CH)
  rel_r = relation.reshape(NW, n_chunks, CH)
  tail_r = tail.reshape(NW, n_chunks, CH)
  out = _tsvec_sc(batch)(head_r, rel_r, tail_r, entity_emb, relation_emb)
  return out.reshape(batch)


# speedup vs baseline: 2.2832x; 1.6668x over previous
import functools

import jax
import jax.numpy as jnp
from jax import lax
from jax.experimental import pallas as pl
from jax.experimental.pallas import tpu as pltpu
from jax.experimental.pallas import tpu_sc as plsc

D = 128
L = 16
NW = 32
CH = 128
NSLOT = 3


def _tsvec_sc(batch):
  b_per_w = batch // NW
  n_chunks = b_per_w // CH
  mesh = plsc.VectorSubcoreMesh(core_axis_name="c", subcore_axis_name="s")

  @functools.partial(
      pl.kernel,
      mesh=mesh,
      compiler_params=pltpu.CompilerParams(needs_layout_passes=False),
      out_type=jax.ShapeDtypeStruct((NW, b_per_w), jnp.float32),
      scratch_types=[
          pltpu.VMEM((n_chunks, CH), jnp.int32),
          pltpu.VMEM((n_chunks, CH), jnp.int32),
          pltpu.VMEM((n_chunks, CH), jnp.int32),
          pltpu.VMEM((NSLOT, CH, D), jnp.float32),
          pltpu.VMEM((NSLOT, CH, D), jnp.float32),
          pltpu.VMEM((b_per_w * L,), jnp.float32),
          pltpu.VMEM((b_per_w,), jnp.float32),
          pltpu.SemaphoreType.DMA((NSLOT,)),
          pltpu.SemaphoreType.DMA((NSLOT,)),
          pltpu.SemaphoreType.DMA((NSLOT,)),
      ],
  )
  def k(head_hbm, rel_hbm, tail_hbm, ent_hbm, relemb_hbm, out_hbm,
        hidx, ridx, tidx, hbuf, tbuf, part, obuf, hsem, tsem, rsem):
    wid = lax.axis_index("s") * 2 + lax.axis_index("c")

    pltpu.sync_copy(head_hbm.at[wid], hidx)
    pltpu.sync_copy(rel_hbm.at[wid], ridx)
    pltpu.sync_copy(tail_hbm.at[wid], tidx)

    def fire_ht(ci):
      slot = ci % NSLOT
      pltpu.async_copy(ent_hbm.at[hidx.at[ci]], hbuf.at[slot], hsem.at[slot])
      pltpu.async_copy(ent_hbm.at[tidx.at[ci]], tbuf.at[slot], tsem.at[slot])

    def wait_h(ci):
      slot = ci % NSLOT
      pltpu.make_async_copy(ent_hbm.at[hidx.at[ci]], hbuf.at[slot],
                            hsem.at[slot]).wait()

    def fire_radd(ci):
      slot = ci % NSLOT
      pltpu.async_copy(relemb_hbm.at[ridx.at[ci]], hbuf.at[slot],
                       rsem.at[slot], add=True)

    def wait_tr(ci):
      slot = ci % NSLOT
      pltpu.make_async_copy(ent_hbm.at[tidx.at[ci]], tbuf.at[slot],
                            tsem.at[slot]).wait()
      pltpu.make_async_copy(relemb_hbm.at[ridx.at[ci]], hbuf.at[slot],
                            rsem.at[slot]).wait()

    fire_ht(0)
    if n_chunks > 1:
      fire_ht(1)
    wait_h(0)
    fire_radd(0)

    for ci in range(n_chunks):
      slot = ci % NSLOT
      if ci + 2 < n_chunks:
        fire_ht(ci + 2)
      if ci + 1 < n_chunks:
        wait_h(ci + 1)
        fire_radd(ci + 1)
      wait_tr(ci)

      U = 4

      @plsc.parallel_loop(0, CH // U)
      def row_body(it):
        base = it * U
        for j in range(U):
          acc = jnp.zeros((L,), jnp.float32)
          for c in range(D // L):
            hv = hbuf[slot, base + j, pl.ds(c * L, L)]
            tv = tbuf[slot, base + j, pl.ds(c * L, L)]
            d = hv - tv
            acc = acc + d * d
          part[pl.ds((ci * CH + base + j) * L, L)] = acc

    lanes = lax.iota(jnp.int32, L)
    for blk in range(b_per_w // L):
      x = jnp.zeros((L,), jnp.float32)
      for j in range(L):
        pv = part[pl.ds((blk * L + j) * L, L)]
        x = jnp.where(lanes == j, jnp.sum(pv), x)
      i = plsc.bitcast(x, jnp.int32)
      i = jnp.int32(0x5F3759DF) - (i >> 1)
      y = plsc.bitcast(i, jnp.float32)
      half_x = 0.5 * x
      for _ in range(3):
        y = y * (1.5 - half_x * y * y)
      obuf[pl.ds(blk * L, L)] = -(x * y)

    pltpu.sync_copy(obuf, out_hbm.at[wid])

  return k


def kernel(head, relation, tail, entity_emb, relation_emb):
  batch = head.shape[0]
  b_per_w = batch // NW
  n_chunks = b_per_w // CH
  head_r = head.reshape(NW, n_chunks, CH)
  rel_r = relation.reshape(NW, n_chunks, CH)
  tail_r = tail.reshape(NW, n_chunks, CH)
  out = _tsvec_sc(batch)(head_r, rel_r, tail_r, entity_emb, relation_emb)
  return out.reshape(batch)
